# Initial kernel scaffold; baseline (speedup 1.0000x reference)
import functools

import jax
import jax.numpy as jnp
from jax import lax
from jax.experimental import pallas as pl
from jax.experimental.pallas import tpu as pltpu
from jax.experimental.pallas import tpu_sc as plsc

N = 10000
NPAD = 10240
E = 320000
IN_CH = 128
HID = 64
OUT = 16
NREL = 2

NC = 2
NS = 16
NW = NC * NS
EPW = E // NW
CH = 80
NCHUNK = EPW // CH
G = CH // 16
TBL = NREL * NPAD
RPT = TBL // NS
CROWS = TBL // 16

BN = 512
GRID = NPAD // BN

_mesh = plsc.VectorSubcoreMesh(
    core_axis_name="c", subcore_axis_name="s", num_cores=NC, num_subcores=NS)


@functools.partial(
    pl.kernel,
    out_type=(
        jax.ShapeDtypeStruct((NC, TBL, HID), jnp.float32),
        jax.ShapeDtypeStruct((NW, CROWS, 16), jnp.float32),
        jax.ShapeDtypeStruct((E // 16, 2, 16), jnp.int32),
    ),
    mesh=_mesh,
    scratch_types=[
        pltpu.VMEM((G, 3, 16), jnp.int32),
        pltpu.VMEM((CH,), jnp.int32),
        pltpu.VMEM((CH,), jnp.int32),
        pltpu.VMEM((G, 2, 16), jnp.int32),
        pltpu.VMEM((CH, HID), jnp.float32),
        pltpu.VMEM((CROWS, 16), jnp.float32),
        pltpu.VMEM_SHARED((TBL, HID), jnp.float32),
        pltpu.SemaphoreType.DMA,
    ],
)
def _sc_layer1(ed_h, h1_h, accp_h, cntp_h, gio_h,
               edv, gidxv, sidxv, giov, rowsv, cntv, acc_sh, sem):
  cid = lax.axis_index("c")
  sid = lax.axis_index("s")
  wid = cid * NS + sid
  zeros16 = jnp.zeros((16,), jnp.float32)
  ones16 = jnp.ones((16,), jnp.float32)

  def zrow(i, _):
    cntv[i] = zeros16
    return 0
  lax.fori_loop(0, CROWS, zrow, 0)

  def zrows(i, _):
    for j in range(HID // 16):
      rowsv[i, pl.ds(j * 16, 16)] = zeros16
    return 0
  lax.fori_loop(0, CH, zrows, 0)

  off0 = sid * RPT
  for k in range(RPT // CH):
    pltpu.sync_copy(rowsv, acc_sh.at[pl.ds(off0 + k * CH, CH)])
  plsc.subcore_barrier()

  gbase = wid * (EPW // 16)

  def chunk(k, _):
    go = pl.multiple_of(gbase + k * G, G)
    pltpu.sync_copy(ed_h.at[pl.ds(go, G)], edv)
    for j in range(G):
      s = edv[j, 0]
      d = edv[j, 1]
      t = edv[j, 2]
      gi = t * NPAD + s
      si = t * NPAD + d
      gidxv[pl.ds(j * 16, 16)] = gi
      sidxv[pl.ds(j * 16, 16)] = si
      giov[j, 0] = gi
      giov[j, 1] = si
      row = lax.shift_right_logical(si, 4)
      col = lax.bitwise_and(si, 15)
      plsc.addupdate_scatter(cntv, [row, col], ones16)
    pltpu.sync_copy(giov, gio_h.at[pl.ds(go, G)])
    pltpu.async_copy(h1_h.at[gidxv], rowsv, sem).wait()
    pltpu.sync_copy(rowsv, acc_sh.at[sidxv], add=True)
    return 0
  lax.fori_loop(0, NCHUNK, chunk, 0)

  plsc.subcore_barrier()
  for k in range(RPT // CH):
    pltpu.sync_copy(acc_sh.at[pl.ds(off0 + k * CH, CH)], rowsv)
    pltpu.sync_copy(rowsv, accp_h.at[cid, pl.ds(off0 + k * CH, CH)])
  pltpu.sync_copy(cntv, cntp_h.at[wid])


@functools.partial(
    pl.kernel,
    out_type=jax.ShapeDtypeStruct((NC, TBL, OUT), jnp.float32),
    mesh=_mesh,
    scratch_types=[
        pltpu.VMEM((G, 2, 16), jnp.int32),
        pltpu.VMEM((CH,), jnp.int32),
        pltpu.VMEM((CH,), jnp.int32),
        pltpu.VMEM((CH, OUT), jnp.float32),
        pltpu.VMEM_SHARED((TBL, OUT), jnp.float32),
        pltpu.SemaphoreType.DMA,
    ],
)
def _sc_layer2(gio_h, h2_h, accp_h, giov, gidxv, sidxv, rowsv, acc_sh, sem):
  cid = lax.axis_index("c")
  sid = lax.axis_index("s")
  wid = cid * NS + sid
  zeros16 = jnp.zeros((16,), jnp.float32)

  def zrows(i, _):
    rowsv[i] = zeros16
    return 0
  lax.fori_loop(0, CH, zrows, 0)

  off0 = sid * RPT
  for k in range(RPT // CH):
    pltpu.sync_copy(rowsv, acc_sh.at[pl.ds(off0 + k * CH, CH)])
  plsc.subcore_barrier()

  gbase = wid * (EPW // 16)

  def chunk(k, _):
    go = pl.multiple_of(gbase + k * G, G)
    pltpu.sync_copy(gio_h.at[pl.ds(go, G)], giov)
    for j in range(G):
      gidxv[pl.ds(j * 16, 16)] = giov[j, 0]
      sidxv[pl.ds(j * 16, 16)] = giov[j, 1]
    pltpu.async_copy(h2_h.at[gidxv], rowsv, sem).wait()
    pltpu.sync_copy(rowsv, acc_sh.at[sidxv], add=True)
    return 0
  lax.fori_loop(0, NCHUNK, chunk, 0)

  plsc.subcore_barrier()
  for k in range(RPT // CH):
    pltpu.sync_copy(acc_sh.at[pl.ds(off0 + k * CH, CH)], rowsv)
    pltpu.sync_copy(rowsv, accp_h.at[cid, pl.ds(off0 + k * CH, CH)])


def _tc_dense1_body(x_ref, root_ref, rel_ref, b_ref, r1_ref, h_ref):
  xb = x_ref[...]
  r1_ref[...] = jnp.dot(xb, root_ref[...],
                        preferred_element_type=jnp.float32) + b_ref[...]
  h0 = jnp.dot(xb, rel_ref[0], preferred_element_type=jnp.float32)
  h1 = jnp.dot(xb, rel_ref[1], preferred_element_type=jnp.float32)
  h_ref[...] = jnp.stack([h0, h1])


def _tc_dense1(xp, root1, rel1, b1):
  return pl.pallas_call(
      _tc_dense1_body,
      grid=(GRID,),
      in_specs=[
          pl.BlockSpec((BN, IN_CH), lambda i: (i, 0)),
          pl.BlockSpec((IN_CH, HID), lambda i: (0, 0)),
          pl.BlockSpec((NREL, IN_CH, HID), lambda i: (0, 0, 0)),
          pl.BlockSpec((1, HID), lambda i: (0, 0)),
      ],
      out_specs=[
          pl.BlockSpec((BN, HID), lambda i: (i, 0)),
          pl.BlockSpec((NREL, BN, HID), lambda i: (0, i, 0)),
      ],
      out_shape=[
          jax.ShapeDtypeStruct((NPAD, HID), jnp.float32),
          jax.ShapeDtypeStruct((NREL, NPAD, HID), jnp.float32),
      ],
  )(xp, root1, rel1, b1)


def _tc_combine1_body(r1_ref, acc_ref, cnt_ref, root2_ref, rel2_ref, b2_ref,
                      r2_ref, h2_ref, inv_ref):
  cnt = jnp.sum(cnt_ref[...], axis=0)
  inv = 1.0 / jnp.maximum(cnt, 1.0)
  inv_ref[...] = inv
  accb = acc_ref[...]
  a0 = accb[0, 0] + accb[1, 0]
  a1 = accb[0, 1] + accb[1, 1]
  out1 = r1_ref[...] + a0 * inv[0][:, None] + a1 * inv[1][:, None]
  out1 = jnp.maximum(out1, 0.0)
  r2_ref[...] = jnp.dot(out1, root2_ref[...],
                        preferred_element_type=jnp.float32) + b2_ref[...]
  h0 = jnp.dot(out1, rel2_ref[0], preferred_element_type=jnp.float32)
  h1 = jnp.dot(out1, rel2_ref[1], preferred_element_type=jnp.float32)
  h2_ref[...] = jnp.stack([h0, h1])


def _tc_combine1(r1, accp, cntp, root2, rel2, b2):
  return pl.pallas_call(
      _tc_combine1_body,
      grid=(GRID,),
      in_specs=[
          pl.BlockSpec((BN, HID), lambda i: (i, 0)),
          pl.BlockSpec((NC, NREL, BN, HID), lambda i: (0, 0, i, 0)),
          pl.BlockSpec((NW, NREL, BN), lambda i: (0, 0, i)),
          pl.BlockSpec((HID, OUT), lambda i: (0, 0)),
          pl.BlockSpec((NREL, HID, OUT), lambda i: (0, 0, 0)),
          pl.BlockSpec((1, OUT), lambda i: (0, 0)),
      ],
      out_specs=[
          pl.BlockSpec((BN, OUT), lambda i: (i, 0)),
          pl.BlockSpec((NREL, BN, OUT), lambda i: (0, i, 0)),
          pl.BlockSpec((NREL, BN), lambda i: (0, i)),
      ],
      out_shape=[
          jax.ShapeDtypeStruct((NPAD, OUT), jnp.float32),
          jax.ShapeDtypeStruct((NREL, NPAD, OUT), jnp.float32),
          jax.ShapeDtypeStruct((NREL, NPAD), jnp.float32),
      ],
  )(r1, accp, cntp, root2, rel2, b2)


def _tc_final_body(r2_ref, acc_ref, inv_ref, out_ref):
  accb = acc_ref[...]
  a0 = accb[0, 0] + accb[1, 0]
  a1 = accb[0, 1] + accb[1, 1]
  inv = inv_ref[...]
  out_ref[...] = r2_ref[...] + a0 * inv[0][:, None] + a1 * inv[1][:, None]


def _tc_final(r2, accp2, inv):
  return pl.pallas_call(
      _tc_final_body,
      grid=(GRID,),
      in_specs=[
          pl.BlockSpec((BN, OUT), lambda i: (i, 0)),
          pl.BlockSpec((NC, NREL, BN, OUT), lambda i: (0, 0, i, 0)),
          pl.BlockSpec((NREL, BN), lambda i: (0, i)),
      ],
      out_specs=pl.BlockSpec((BN, OUT), lambda i: (i, 0)),
      out_shape=jax.ShapeDtypeStruct((NPAD, OUT), jnp.float32),
  )(r2, accp2, inv)


@jax.jit
def kernel(x, edge_index, edge_type, root1, rel1, b1, root2, rel2, b2):
  src = edge_index[0]
  dst = edge_index[1]
  ed = jnp.concatenate([
      src.reshape(E // 16, 1, 16),
      dst.reshape(E // 16, 1, 16),
      edge_type.reshape(E // 16, 1, 16),
  ], axis=1)
  xp = jnp.pad(x, ((0, NPAD - N), (0, 0)))

  r1, h1 = _tc_dense1(xp, root1, rel1, b1.reshape(1, HID))
  accp, cntp, gio = _sc_layer1(ed, h1.reshape(TBL, HID))
  r2, h2, inv = _tc_combine1(
      r1,
      accp.reshape(NC, NREL, NPAD, HID),
      cntp.reshape(NW, NREL, NPAD),
      root2, rel2, b2.reshape(1, OUT))
  accp2 = _sc_layer2(gio, h2.reshape(TBL, OUT))
  out = _tc_final(r2, accp2.reshape(NC, NREL, NPAD, OUT), inv)
  return out[:N]


# speedup vs baseline: 9.0908x; 9.0908x over previous
import functools

import jax
import jax.numpy as jnp
from jax import lax
from jax.experimental import pallas as pl
from jax.experimental.pallas import tpu as pltpu
from jax.experimental.pallas import tpu_sc as plsc

N = 10000
NPAD = 10240
E = 320000
IN_CH = 128
HID = 64
OUT = 16
NREL = 2

NC = 2
NS = 16
NW = NC * NS
EPW = E // NW
CH = 80
NCHUNK = EPW // CH
G = CH // 16
TBL = NREL * NPAD
RPT = TBL // NS
CROWS = TBL // 16

BN = 512
GRID = NPAD // BN

_mesh = plsc.VectorSubcoreMesh(
    core_axis_name="c", subcore_axis_name="s", num_cores=NC, num_subcores=NS)


@functools.partial(
    pl.kernel,
    out_type=(
        jax.ShapeDtypeStruct((NC, TBL, HID), jnp.float32),
        jax.ShapeDtypeStruct((NW, CROWS, 16), jnp.float32),
        jax.ShapeDtypeStruct((E,), jnp.int32),
        jax.ShapeDtypeStruct((E,), jnp.int32),
    ),
    mesh=_mesh,
    scratch_types=[
        pltpu.VMEM((CH,), jnp.int32),
        pltpu.VMEM((CH,), jnp.int32),
        pltpu.VMEM((CH,), jnp.int32),
        pltpu.VMEM((CH,), jnp.int32),
        pltpu.VMEM((CH,), jnp.int32),
        pltpu.VMEM((CH, HID), jnp.float32),
        pltpu.VMEM((CROWS, 16), jnp.float32),
        pltpu.VMEM_SHARED((TBL, HID), jnp.float32),
        pltpu.SemaphoreType.DMA,
    ],
    compiler_params=pltpu.CompilerParams(needs_layout_passes=False, use_tc_tiling_on_sc=False),
)
def _sc_layer1(src_h, dst_h, et_h, h1_h, z64_h, z16_h,
               accp_h, cntp_h, gidx_h, sidx_h,
               srcv, dstv, etv, gidxv, sidxv, rowsv, cntv, acc_sh, sem):
  cid = lax.axis_index("c")
  sid = lax.axis_index("s")
  wid = cid * NS + sid
  ones16 = jnp.ones((16,), jnp.float32)

  pltpu.sync_copy(z16_h, cntv)
  pltpu.sync_copy(z64_h, rowsv)
  off0 = sid * RPT
  for k in range(RPT // CH):
    pltpu.sync_copy(rowsv, acc_sh.at[pl.ds(off0 + k * CH, CH)])
  plsc.subcore_barrier()

  base = wid * EPW

  def chunk(k, _):
    eo = pl.multiple_of(base + k * CH, CH)
    pltpu.sync_copy(src_h.at[pl.ds(eo, CH)], srcv)
    pltpu.sync_copy(dst_h.at[pl.ds(eo, CH)], dstv)
    pltpu.sync_copy(et_h.at[pl.ds(eo, CH)], etv)
    for j in range(G):
      s = srcv[pl.ds(j * 16, 16)]
      d = dstv[pl.ds(j * 16, 16)]
      t = etv[pl.ds(j * 16, 16)]
      gi = t * NPAD + s
      si = t * NPAD + d
      gidxv[pl.ds(j * 16, 16)] = gi
      sidxv[pl.ds(j * 16, 16)] = si
      row = lax.shift_right_logical(si, 4)
      col = lax.bitwise_and(si, 15)
      plsc.addupdate_scatter(cntv, [row, col], ones16)
    pltpu.sync_copy(gidxv, gidx_h.at[pl.ds(eo, CH)])
    pltpu.sync_copy(sidxv, sidx_h.at[pl.ds(eo, CH)])
    pltpu.async_copy(h1_h.at[gidxv], rowsv, sem).wait()
    pltpu.sync_copy(rowsv, acc_sh.at[sidxv], add=True)
    return 0
  lax.fori_loop(0, NCHUNK, chunk, 0)

  plsc.subcore_barrier()
  for k in range(RPT // CH):
    pltpu.sync_copy(acc_sh.at[pl.ds(off0 + k * CH, CH)], rowsv)
    pltpu.sync_copy(rowsv, accp_h.at[cid, pl.ds(off0 + k * CH, CH)])
  pltpu.sync_copy(cntv, cntp_h.at[wid])


@functools.partial(
    pl.kernel,
    out_type=jax.ShapeDtypeStruct((NC, TBL, OUT), jnp.float32),
    mesh=_mesh,
    scratch_types=[
        pltpu.VMEM((CH,), jnp.int32),
        pltpu.VMEM((CH,), jnp.int32),
        pltpu.VMEM((CH, OUT), jnp.float32),
        pltpu.VMEM_SHARED((TBL, OUT), jnp.float32),
        pltpu.SemaphoreType.DMA,
    ],
    compiler_params=pltpu.CompilerParams(needs_layout_passes=False, use_tc_tiling_on_sc=False),
)
def _sc_layer2(gidx_h, sidx_h, h2_h, z16_h, accp_h,
               gidxv, sidxv, rowsv, acc_sh, sem):
  cid = lax.axis_index("c")
  sid = lax.axis_index("s")
  wid = cid * NS + sid

  pltpu.sync_copy(z16_h.at[pl.ds(0, CH)], rowsv)
  off0 = sid * RPT
  for k in range(RPT // CH):
    pltpu.sync_copy(rowsv, acc_sh.at[pl.ds(off0 + k * CH, CH)])
  plsc.subcore_barrier()

  base = wid * EPW

  def chunk(k, _):
    eo = pl.multiple_of(base + k * CH, CH)
    pltpu.sync_copy(gidx_h.at[pl.ds(eo, CH)], gidxv)
    pltpu.sync_copy(sidx_h.at[pl.ds(eo, CH)], sidxv)
    pltpu.async_copy(h2_h.at[gidxv], rowsv, sem).wait()
    pltpu.sync_copy(rowsv, acc_sh.at[sidxv], add=True)
    return 0
  lax.fori_loop(0, NCHUNK, chunk, 0)

  plsc.subcore_barrier()
  for k in range(RPT // CH):
    pltpu.sync_copy(acc_sh.at[pl.ds(off0 + k * CH, CH)], rowsv)
    pltpu.sync_copy(rowsv, accp_h.at[cid, pl.ds(off0 + k * CH, CH)])


def _tc_dense1_body(x_ref, root_ref, rel_ref, b_ref, r1_ref, h_ref):
  xb = x_ref[...]
  r1_ref[...] = jnp.dot(xb, root_ref[...],
                        preferred_element_type=jnp.float32) + b_ref[...]
  h0 = jnp.dot(xb, rel_ref[0], preferred_element_type=jnp.float32)
  h1 = jnp.dot(xb, rel_ref[1], preferred_element_type=jnp.float32)
  h_ref[...] = jnp.stack([h0, h1])


def _tc_dense1(xp, root1, rel1, b1):
  return pl.pallas_call(
      _tc_dense1_body,
      grid=(GRID,),
      in_specs=[
          pl.BlockSpec((BN, IN_CH), lambda i: (i, 0)),
          pl.BlockSpec((IN_CH, HID), lambda i: (0, 0)),
          pl.BlockSpec((NREL, IN_CH, HID), lambda i: (0, 0, 0)),
          pl.BlockSpec((1, HID), lambda i: (0, 0)),
      ],
      out_specs=[
          pl.BlockSpec((BN, HID), lambda i: (i, 0)),
          pl.BlockSpec((NREL, BN, HID), lambda i: (0, i, 0)),
      ],
      out_shape=[
          jax.ShapeDtypeStruct((NPAD, HID), jnp.float32),
          jax.ShapeDtypeStruct((NREL, NPAD, HID), jnp.float32),
      ],
  )(xp, root1, rel1, b1)


def _tc_combine1_body(r1_ref, acc_ref, cnt_ref, root2_ref, rel2_ref, b2_ref,
                      r2_ref, h2_ref, inv_ref):
  cnt = jnp.sum(cnt_ref[...], axis=0)
  inv = 1.0 / jnp.maximum(cnt, 1.0)
  inv_ref[...] = inv
  accb = acc_ref[...]
  a0 = accb[0, 0] + accb[1, 0]
  a1 = accb[0, 1] + accb[1, 1]
  out1 = r1_ref[...] + a0 * inv[0][:, None] + a1 * inv[1][:, None]
  out1 = jnp.maximum(out1, 0.0)
  r2_ref[...] = jnp.dot(out1, root2_ref[...],
                        preferred_element_type=jnp.float32) + b2_ref[...]
  h0 = jnp.dot(out1, rel2_ref[0], preferred_element_type=jnp.float32)
  h1 = jnp.dot(out1, rel2_ref[1], preferred_element_type=jnp.float32)
  h2_ref[...] = jnp.stack([h0, h1])


def _tc_combine1(r1, accp, cntp, root2, rel2, b2):
  return pl.pallas_call(
      _tc_combine1_body,
      grid=(GRID,),
      in_specs=[
          pl.BlockSpec((BN, HID), lambda i: (i, 0)),
          pl.BlockSpec((NC, NREL, BN, HID), lambda i: (0, 0, i, 0)),
          pl.BlockSpec((NW, NREL, BN), lambda i: (0, 0, i)),
          pl.BlockSpec((HID, OUT), lambda i: (0, 0)),
          pl.BlockSpec((NREL, HID, OUT), lambda i: (0, 0, 0)),
          pl.BlockSpec((1, OUT), lambda i: (0, 0)),
      ],
      out_specs=[
          pl.BlockSpec((BN, OUT), lambda i: (i, 0)),
          pl.BlockSpec((NREL, BN, OUT), lambda i: (0, i, 0)),
          pl.BlockSpec((NREL, BN), lambda i: (0, i)),
      ],
      out_shape=[
          jax.ShapeDtypeStruct((NPAD, OUT), jnp.float32),
          jax.ShapeDtypeStruct((NREL, NPAD, OUT), jnp.float32),
          jax.ShapeDtypeStruct((NREL, NPAD), jnp.float32),
      ],
  )(r1, accp, cntp, root2, rel2, b2)


def _tc_final_body(r2_ref, acc_ref, inv_ref, out_ref):
  accb = acc_ref[...]
  a0 = accb[0, 0] + accb[1, 0]
  a1 = accb[0, 1] + accb[1, 1]
  inv = inv_ref[...]
  out_ref[...] = r2_ref[...] + a0 * inv[0][:, None] + a1 * inv[1][:, None]


def _tc_final(r2, accp2, inv):
  return pl.pallas_call(
      _tc_final_body,
      grid=(GRID,),
      in_specs=[
          pl.BlockSpec((BN, OUT), lambda i: (i, 0)),
          pl.BlockSpec((NC, NREL, BN, OUT), lambda i: (0, 0, i, 0)),
          pl.BlockSpec((NREL, BN), lambda i: (0, i)),
      ],
      out_specs=pl.BlockSpec((BN, OUT), lambda i: (i, 0)),
      out_shape=jax.ShapeDtypeStruct((NPAD, OUT), jnp.float32),
  )(r2, accp2, inv)


@jax.jit
def kernel(x, edge_index, edge_type, root1, rel1, b1, root2, rel2, b2):
  src = edge_index[0]
  dst = edge_index[1]
  xp = jnp.pad(x, ((0, NPAD - N), (0, 0)))
  z64 = jnp.zeros((CH, HID), jnp.float32)
  z16 = jnp.zeros((CROWS, 16), jnp.float32)

  r1, h1 = _tc_dense1(xp, root1, rel1, b1.reshape(1, HID))
  accp, cntp, gidx, sidx = _sc_layer1(
      src, dst, edge_type, h1.reshape(TBL, HID), z64, z16)
  r2, h2, inv = _tc_combine1(
      r1,
      accp.reshape(NC, NREL, NPAD, HID),
      cntp.reshape(NW, NREL, NPAD),
      root2, rel2, b2.reshape(1, OUT))
  accp2 = _sc_layer2(gidx, sidx, h2.reshape(TBL, OUT), z16)
  out = _tc_final(r2, accp2.reshape(NC, NREL, NPAD, OUT), inv)
  return out[:N]


# speedup vs baseline: 19.9257x; 2.1919x over previous
import functools

import jax
import jax.numpy as jnp
from jax import lax
from jax.experimental import pallas as pl
from jax.experimental.pallas import tpu as pltpu
from jax.experimental.pallas import tpu_sc as plsc

N = 10000
NPAD = 10240
E = 320000
IN_CH = 128
HID = 64
OUT = 16
NREL = 2

NC = 2
NS = 16
NW = NC * NS
EPW = E // NW
CH = 80
NCHUNK = EPW // CH
G = CH // 16
TBL = NREL * NPAD
RPT = TBL // NS
CROWS = TBL // 16

BN = 512
GRID = NPAD // BN

_mesh = plsc.VectorSubcoreMesh(
    core_axis_name="c", subcore_axis_name="s", num_cores=NC, num_subcores=NS)


NBUF = 5
NOUT = NCHUNK // NBUF

_sc1_scratch = (
    [pltpu.VMEM((CH,), jnp.int32) for _ in range(NBUF)]
    + [pltpu.VMEM((CH,), jnp.int32) for _ in range(NBUF)]
    + [pltpu.VMEM((CH,), jnp.int32) for _ in range(NBUF)]
    + [pltpu.VMEM((CH,), jnp.int32) for _ in range(NBUF)]
    + [pltpu.VMEM((CH,), jnp.int32) for _ in range(NBUF)]
    + [pltpu.VMEM((CH, HID), jnp.float32) for _ in range(NBUF)]
    + [pltpu.VMEM((CROWS, 16), jnp.float32),
       pltpu.VMEM_SHARED((TBL, HID), jnp.float32)]
    + [pltpu.SemaphoreType.DMA for _ in range(3 * NBUF + 1)]
)


@functools.partial(
    pl.kernel,
    out_type=(
        jax.ShapeDtypeStruct((NC, TBL, HID), jnp.float32),
        jax.ShapeDtypeStruct((NW, CROWS, 16), jnp.float32),
        jax.ShapeDtypeStruct((E,), jnp.int32),
        jax.ShapeDtypeStruct((E,), jnp.int32),
    ),
    mesh=_mesh,
    scratch_types=_sc1_scratch,
    compiler_params=pltpu.CompilerParams(needs_layout_passes=False,
                                         use_tc_tiling_on_sc=False),
)
def _sc_layer1(src_h, dst_h, et_h, h1_h, z64_h, z16_h,
               accp_h, cntp_h, gidx_h, sidx_h, *scr):
  srcv = scr[0:NBUF]
  dstv = scr[NBUF:2 * NBUF]
  etv = scr[2 * NBUF:3 * NBUF]
  gidxv = scr[3 * NBUF:4 * NBUF]
  sidxv = scr[4 * NBUF:5 * NBUF]
  rowsv = scr[5 * NBUF:6 * NBUF]
  cntv = scr[6 * NBUF]
  acc_sh = scr[6 * NBUF + 1]
  lsem = scr[6 * NBUF + 2:6 * NBUF + 2 + NBUF]
  gsem = scr[6 * NBUF + 2 + NBUF:6 * NBUF + 2 + 2 * NBUF]
  ssem = scr[6 * NBUF + 2 + 2 * NBUF:6 * NBUF + 2 + 3 * NBUF]
  wsem = scr[6 * NBUF + 2 + 3 * NBUF]

  cid = lax.axis_index("c")
  sid = lax.axis_index("s")
  wid = cid * NS + sid
  ones16 = jnp.ones((16,), jnp.float32)

  pltpu.sync_copy(z16_h, cntv)
  pltpu.sync_copy(z64_h, rowsv[0])
  off0 = sid * RPT
  for k in range(RPT // CH):
    pltpu.sync_copy(rowsv[0], acc_sh.at[pl.ds(off0 + k * CH, CH)])
  plsc.subcore_barrier()

  base = wid * EPW

  def outer(ko, _):
    eo0 = pl.multiple_of(base + ko * (NBUF * CH), CH)
    lds = []
    for b in range(NBUF):
      eo = eo0 + b * CH
      lds.append((
          pltpu.async_copy(src_h.at[pl.ds(eo, CH)], srcv[b], lsem[b]),
          pltpu.async_copy(dst_h.at[pl.ds(eo, CH)], dstv[b], lsem[b]),
          pltpu.async_copy(et_h.at[pl.ds(eo, CH)], etv[b], lsem[b]),
      ))
    gds = []
    wds = []
    for b in range(NBUF):
      for d in lds[b]:
        d.wait()
      for j in range(G):
        s = srcv[b][pl.ds(j * 16, 16)]
        d_ = dstv[b][pl.ds(j * 16, 16)]
        t = etv[b][pl.ds(j * 16, 16)]
        gi = t * NPAD + s
        si = t * NPAD + d_
        gidxv[b][pl.ds(j * 16, 16)] = gi
        sidxv[b][pl.ds(j * 16, 16)] = si
        row = lax.shift_right_logical(si, 4)
        col = lax.bitwise_and(si, 15)
        plsc.addupdate_scatter(cntv, [row, col], ones16)
      eo = eo0 + b * CH
      wds.append(pltpu.async_copy(gidxv[b], gidx_h.at[pl.ds(eo, CH)], wsem))
      wds.append(pltpu.async_copy(sidxv[b], sidx_h.at[pl.ds(eo, CH)], wsem))
      gds.append(pltpu.async_copy(h1_h.at[gidxv[b]], rowsv[b], gsem[b]))
    sds = []
    for b in range(NBUF):
      gds[b].wait()
      sds.append(pltpu.async_copy(rowsv[b], acc_sh.at[sidxv[b]], ssem[b],
                                  add=True))
    for b in range(NBUF):
      sds[b].wait()
    for d in wds:
      d.wait()
    return 0
  lax.fori_loop(0, NOUT, outer, 0)

  plsc.subcore_barrier()
  for k in range(RPT // CH):
    pltpu.sync_copy(acc_sh.at[pl.ds(off0 + k * CH, CH)], rowsv[0])
    pltpu.sync_copy(rowsv[0], accp_h.at[cid, pl.ds(off0 + k * CH, CH)])
  pltpu.sync_copy(cntv, cntp_h.at[wid])


_sc2_scratch = (
    [pltpu.VMEM((CH,), jnp.int32) for _ in range(NBUF)]
    + [pltpu.VMEM((CH,), jnp.int32) for _ in range(NBUF)]
    + [pltpu.VMEM((CH, OUT), jnp.float32) for _ in range(NBUF)]
    + [pltpu.VMEM_SHARED((TBL, OUT), jnp.float32)]
    + [pltpu.SemaphoreType.DMA for _ in range(3 * NBUF)]
)


@functools.partial(
    pl.kernel,
    out_type=jax.ShapeDtypeStruct((NC, TBL, OUT), jnp.float32),
    mesh=_mesh,
    scratch_types=_sc2_scratch,
    compiler_params=pltpu.CompilerParams(needs_layout_passes=False,
                                         use_tc_tiling_on_sc=False),
)
def _sc_layer2(gidx_h, sidx_h, h2_h, z16_h, accp_h, *scr):
  gidxv = scr[0:NBUF]
  sidxv = scr[NBUF:2 * NBUF]
  rowsv = scr[2 * NBUF:3 * NBUF]
  acc_sh = scr[3 * NBUF]
  lsem = scr[3 * NBUF + 1:3 * NBUF + 1 + NBUF]
  gsem = scr[3 * NBUF + 1 + NBUF:3 * NBUF + 1 + 2 * NBUF]
  ssem = scr[3 * NBUF + 1 + 2 * NBUF:3 * NBUF + 1 + 3 * NBUF]

  cid = lax.axis_index("c")
  sid = lax.axis_index("s")
  wid = cid * NS + sid

  pltpu.sync_copy(z16_h.at[pl.ds(0, CH)], rowsv[0])
  off0 = sid * RPT
  for k in range(RPT // CH):
    pltpu.sync_copy(rowsv[0], acc_sh.at[pl.ds(off0 + k * CH, CH)])
  plsc.subcore_barrier()

  base = wid * EPW

  def outer(ko, _):
    eo0 = pl.multiple_of(base + ko * (NBUF * CH), CH)
    lds = []
    for b in range(NBUF):
      eo = eo0 + b * CH
      lds.append((
          pltpu.async_copy(gidx_h.at[pl.ds(eo, CH)], gidxv[b], lsem[b]),
          pltpu.async_copy(sidx_h.at[pl.ds(eo, CH)], sidxv[b], lsem[b]),
      ))
    gds = []
    for b in range(NBUF):
      for d in lds[b]:
        d.wait()
      gds.append(pltpu.async_copy(h2_h.at[gidxv[b]], rowsv[b], gsem[b]))
    sds = []
    for b in range(NBUF):
      gds[b].wait()
      sds.append(pltpu.async_copy(rowsv[b], acc_sh.at[sidxv[b]], ssem[b],
                                  add=True))
    for b in range(NBUF):
      sds[b].wait()
    return 0
  lax.fori_loop(0, NOUT, outer, 0)

  plsc.subcore_barrier()
  for k in range(RPT // CH):
    pltpu.sync_copy(acc_sh.at[pl.ds(off0 + k * CH, CH)], rowsv[0])
    pltpu.sync_copy(rowsv[0], accp_h.at[cid, pl.ds(off0 + k * CH, CH)])


def _tc_dense1_body(x_ref, root_ref, rel_ref, b_ref, r1_ref, h_ref):
  xb = x_ref[...]
  r1_ref[...] = jnp.dot(xb, root_ref[...],
                        preferred_element_type=jnp.float32) + b_ref[...]
  h0 = jnp.dot(xb, rel_ref[0], preferred_element_type=jnp.float32)
  h1 = jnp.dot(xb, rel_ref[1], preferred_element_type=jnp.float32)
  h_ref[...] = jnp.stack([h0, h1])


def _tc_dense1(xp, root1, rel1, b1):
  return pl.pallas_call(
      _tc_dense1_body,
      grid=(GRID,),
      in_specs=[
          pl.BlockSpec((BN, IN_CH), lambda i: (i, 0)),
          pl.BlockSpec((IN_CH, HID), lambda i: (0, 0)),
          pl.BlockSpec((NREL, IN_CH, HID), lambda i: (0, 0, 0)),
          pl.BlockSpec((1, HID), lambda i: (0, 0)),
      ],
      out_specs=[
          pl.BlockSpec((BN, HID), lambda i: (i, 0)),
          pl.BlockSpec((NREL, BN, HID), lambda i: (0, i, 0)),
      ],
      out_shape=[
          jax.ShapeDtypeStruct((NPAD, HID), jnp.float32),
          jax.ShapeDtypeStruct((NREL, NPAD, HID), jnp.float32),
      ],
  )(xp, root1, rel1, b1)


def _tc_combine1_body(r1_ref, acc_ref, cnt_ref, root2_ref, rel2_ref, b2_ref,
                      r2_ref, h2_ref, inv_ref):
  cnt = jnp.sum(cnt_ref[...], axis=0)
  inv = 1.0 / jnp.maximum(cnt, 1.0)
  inv_ref[...] = inv
  accb = acc_ref[...]
  a0 = accb[0, 0] + accb[1, 0]
  a1 = accb[0, 1] + accb[1, 1]
  out1 = r1_ref[...] + a0 * inv[0][:, None] + a1 * inv[1][:, None]
  out1 = jnp.maximum(out1, 0.0)
  r2_ref[...] = jnp.dot(out1, root2_ref[...],
                        preferred_element_type=jnp.float32) + b2_ref[...]
  h0 = jnp.dot(out1, rel2_ref[0], preferred_element_type=jnp.float32)
  h1 = jnp.dot(out1, rel2_ref[1], preferred_element_type=jnp.float32)
  h2_ref[...] = jnp.stack([h0, h1])


def _tc_combine1(r1, accp, cntp, root2, rel2, b2):
  return pl.pallas_call(
      _tc_combine1_body,
      grid=(GRID,),
      in_specs=[
          pl.BlockSpec((BN, HID), lambda i: (i, 0)),
          pl.BlockSpec((NC, NREL, BN, HID), lambda i: (0, 0, i, 0)),
          pl.BlockSpec((NW, NREL, BN), lambda i: (0, 0, i)),
          pl.BlockSpec((HID, OUT), lambda i: (0, 0)),
          pl.BlockSpec((NREL, HID, OUT), lambda i: (0, 0, 0)),
          pl.BlockSpec((1, OUT), lambda i: (0, 0)),
      ],
      out_specs=[
          pl.BlockSpec((BN, OUT), lambda i: (i, 0)),
          pl.BlockSpec((NREL, BN, OUT), lambda i: (0, i, 0)),
          pl.BlockSpec((NREL, BN), lambda i: (0, i)),
      ],
      out_shape=[
          jax.ShapeDtypeStruct((NPAD, OUT), jnp.float32),
          jax.ShapeDtypeStruct((NREL, NPAD, OUT), jnp.float32),
          jax.ShapeDtypeStruct((NREL, NPAD), jnp.float32),
      ],
  )(r1, accp, cntp, root2, rel2, b2)


def _tc_final_body(r2_ref, acc_ref, inv_ref, out_ref):
  accb = acc_ref[...]
  a0 = accb[0, 0] + accb[1, 0]
  a1 = accb[0, 1] + accb[1, 1]
  inv = inv_ref[...]
  out_ref[...] = r2_ref[...] + a0 * inv[0][:, None] + a1 * inv[1][:, None]


def _tc_final(r2, accp2, inv):
  return pl.pallas_call(
      _tc_final_body,
      grid=(GRID,),
      in_specs=[
          pl.BlockSpec((BN, OUT), lambda i: (i, 0)),
          pl.BlockSpec((NC, NREL, BN, OUT), lambda i: (0, 0, i, 0)),
          pl.BlockSpec((NREL, BN), lambda i: (0, i)),
      ],
      out_specs=pl.BlockSpec((BN, OUT), lambda i: (i, 0)),
      out_shape=jax.ShapeDtypeStruct((NPAD, OUT), jnp.float32),
  )(r2, accp2, inv)


@jax.jit
def kernel(x, edge_index, edge_type, root1, rel1, b1, root2, rel2, b2):
  src = edge_index[0]
  dst = edge_index[1]
  xp = jnp.pad(x, ((0, NPAD - N), (0, 0)))
  z64 = jnp.zeros((CH, HID), jnp.float32)
  z16 = jnp.zeros((CROWS, 16), jnp.float32)

  r1, h1 = _tc_dense1(xp, root1, rel1, b1.reshape(1, HID))
  accp, cntp, gidx, sidx = _sc_layer1(
      src, dst, edge_type, h1.reshape(TBL, HID), z64, z16)
  r2, h2, inv = _tc_combine1(
      r1,
      accp.reshape(NC, NREL, NPAD, HID),
      cntp.reshape(NW, NREL, NPAD),
      root2, rel2, b2.reshape(1, OUT))
  accp2 = _sc_layer2(gidx, sidx, h2.reshape(TBL, OUT), z16)
  out = _tc_final(r2, accp2.reshape(NC, NREL, NPAD, OUT), inv)
  return out[:N]
